# Initial kernel scaffold; baseline (speedup 1.0000x reference)
import jax
import jax.numpy as jnp
from jax.experimental import pallas as pl


def kernel(input_indices, embedding_matrix):
    raise NotImplementedError("write your pallas kernel here")


# speedup vs baseline: 6.3903x; 6.3903x over previous
import functools
import jax
import jax.numpy as jnp
from jax import lax
from jax.experimental import pallas as pl
from jax.experimental.pallas import tpu as pltpu
from jax.experimental.pallas import tpu_sc as plsc

VOCAB = 100004
EMBED_DIM = 128
BATCH = 4096
HIST_LEN = 200

_TOTAL = BATCH * HIST_LEN
_IDX_COLS = 128
_IDX_ROWS = _TOTAL // _IDX_COLS
_NW = 32
_ROWS_PER_W = _IDX_ROWS // _NW


def _gather_body(idx_hbm, table_hbm, out_hbm, idx_v, rows_v, sem_g, sem_s):
    wid = lax.axis_index("s") * 2 + lax.axis_index("c")
    row_base = wid * _ROWS_PER_W

    pltpu.sync_copy(idx_hbm.at[pl.ds(row_base, _ROWS_PER_W)], idx_v)

    @pl.loop(0, _ROWS_PER_W)
    def _(g):
        pltpu.async_copy(table_hbm.at[idx_v.at[g]], rows_v, sem_g).wait()
        pltpu.async_copy(
            rows_v, out_hbm.at[pl.ds((row_base + g) * _IDX_COLS, _IDX_COLS)],
            sem_s,
        ).wait()


def kernel(input_indices, embedding_matrix):
    idx2d = input_indices.reshape(_IDX_ROWS, _IDX_COLS)

    mesh = plsc.VectorSubcoreMesh(core_axis_name="c", subcore_axis_name="s")
    out_flat = pl.kernel(
        _gather_body,
        mesh=mesh,
        out_type=jax.ShapeDtypeStruct((_TOTAL, EMBED_DIM), jnp.float32),
        scratch_types=[
            pltpu.VMEM((_ROWS_PER_W, _IDX_COLS), jnp.int32),
            pltpu.VMEM((_IDX_COLS, EMBED_DIM), jnp.float32),
            pltpu.SemaphoreType.DMA,
            pltpu.SemaphoreType.DMA,
        ],
    )(idx2d, embedding_matrix)

    return out_flat.reshape(BATCH, HIST_LEN, EMBED_DIM)


# speedup vs baseline: 9.2886x; 1.4535x over previous
import functools
import jax
import jax.numpy as jnp
from jax import lax
from jax.experimental import pallas as pl
from jax.experimental.pallas import tpu as pltpu
from jax.experimental.pallas import tpu_sc as plsc

VOCAB = 100004
EMBED_DIM = 128
BATCH = 4096
HIST_LEN = 200

_TOTAL = BATCH * HIST_LEN
_IDX_COLS = 128
_IDX_ROWS = _TOTAL // _IDX_COLS
_NW = 32
_ROWS_PER_W = _IDX_ROWS // _NW


def _gather_body(idx_hbm, table_hbm, out_hbm, idx_v, rows0, rows1, sem0, sem1):
    wid = lax.axis_index("s") * 2 + lax.axis_index("c")
    row_base = wid * _ROWS_PER_W

    pltpu.sync_copy(idx_hbm.at[pl.ds(row_base, _ROWS_PER_W)], idx_v)

    rows = (rows0, rows1)
    sems = (sem0, sem1)

    def gather_start(g, b):
        pltpu.async_copy(table_hbm.at[idx_v.at[g]], rows[b], sems[b])

    def store_sync(g, b):
        pltpu.sync_copy(
            rows[b], out_hbm.at[pl.ds((row_base + g) * _IDX_COLS, _IDX_COLS)]
        )

    def wait_gather(b):
        pltpu.make_async_copy(table_hbm.at[idx_v.at[0]], rows[b], sems[b]).wait()

    gather_start(0, 0)

    @pl.loop(0, _ROWS_PER_W - 2, step=2)
    def _(g0):
        for b in range(2):
            g = g0 + b
            gather_start(g + 1, 1 - b)
            wait_gather(b)
            store_sync(g, b)

    g_tail = _ROWS_PER_W - 2
    gather_start(g_tail + 1, 1)
    wait_gather(0)
    store_sync(g_tail, 0)
    wait_gather(1)
    store_sync(g_tail + 1, 1)


def kernel(input_indices, embedding_matrix):
    idx2d = input_indices.reshape(_IDX_ROWS, _IDX_COLS)

    mesh = plsc.VectorSubcoreMesh(core_axis_name="c", subcore_axis_name="s")
    out_flat = pl.kernel(
        _gather_body,
        mesh=mesh,
        out_type=jax.ShapeDtypeStruct((_TOTAL, EMBED_DIM), jnp.float32),
        scratch_types=[
            pltpu.VMEM((_ROWS_PER_W, _IDX_COLS), jnp.int32),
            pltpu.VMEM((_IDX_COLS, EMBED_DIM), jnp.float32),
            pltpu.VMEM((_IDX_COLS, EMBED_DIM), jnp.float32),
            pltpu.SemaphoreType.DMA,
            pltpu.SemaphoreType.DMA,
        ],
    )(idx2d, embedding_matrix)

    return out_flat.reshape(BATCH, HIST_LEN, EMBED_DIM)
